# Initial kernel scaffold; baseline (speedup 1.0000x reference)
import jax
import jax.numpy as jnp
from jax.experimental import pallas as pl


def kernel(entity, relation, fc_W, fc_b, src_ids0, edge_src0, etype0, src_ids1, edge_src1, etype1):
    raise NotImplementedError("write your pallas kernel here")


# speedup vs baseline: 4.1735x; 4.1735x over previous
import jax
import jax.numpy as jnp
from jax import lax
from jax.experimental import pallas as pl
from jax.experimental.pallas import tpu as pltpu
from jax.experimental.pallas import tpu_sc as plsc

HIDDEN = 128
NUM_RELS = 64
NUM_TYPES = 16
DEG = 32
N_DST0 = 10000
N_DST1 = 10000
N_SRC0 = 20000
N_SRC1 = 10000
E0 = N_DST0 * DEG
E1 = N_DST1 * DEG
BETA = 0.3

NW = 32
CH_D0 = 8
CH_E = CH_D0 * DEG
NCHUNK0 = N_DST0 // CH_D0
KMAX0 = (NCHUNK0 + NW - 1) // NW
NCHUNK1 = E1 // CH_E
KMAX1 = (NCHUNK1 + NW - 1) // NW

TILE0 = 400
TILE1 = 80


def _sc1_body(entity, src_ids0, edge_src0, aggE,
              src_tab, echunk, idxbuf, rows, accbuf, sem):
    wid = lax.axis_index("s") * 2 + lax.axis_index("c")
    pltpu.sync_copy(src_ids0, src_tab)

    def chunk_body(k, carry):
        c = wid + k * NW

        @pl.when(c < NCHUNK0)
        def _():
            pltpu.sync_copy(edge_src0.at[pl.ds(c * CH_E, CH_E)], echunk)
            for j in range(CH_E // 16):
                ev = echunk[pl.ds(j * 16, 16)]
                idxbuf[pl.ds(j * 16, 16)] = plsc.load_gather(src_tab, [ev])
            pltpu.async_copy(entity.at[idxbuf], rows, sem).wait()
            for d in range(CH_D0):
                def acc_body(kk, accs):
                    return tuple(
                        accs[j] + rows[d * DEG + kk, pl.ds(j * 16, 16)]
                        for j in range(HIDDEN // 16))
                accs = lax.fori_loop(
                    0, DEG, acc_body,
                    tuple(jnp.zeros((16,), jnp.float32)
                          for _ in range(HIDDEN // 16)),
                    unroll=4)
                for j in range(HIDDEN // 16):
                    accbuf[d, pl.ds(j * 16, 16)] = accs[j]
            pltpu.sync_copy(accbuf, aggE.at[pl.ds(c * CH_D0, CH_D0)])
        return carry

    lax.fori_loop(0, KMAX0, chunk_body, 0)


def _sc1(entity, src_ids0, edge_src0):
    mesh = plsc.VectorSubcoreMesh(core_axis_name="c", subcore_axis_name="s")
    f = pl.kernel(
        _sc1_body,
        out_type=jax.ShapeDtypeStruct((N_DST0, HIDDEN), jnp.float32),
        mesh=mesh,
        scratch_types=[
            pltpu.VMEM((N_SRC0,), jnp.int32),
            pltpu.VMEM((CH_E,), jnp.int32),
            pltpu.VMEM((CH_E,), jnp.int32),
            pltpu.VMEM((CH_E, HIDDEN), jnp.float32),
            pltpu.VMEM((CH_D0, HIDDEN), jnp.float32),
            pltpu.SemaphoreType.DMA,
        ],
        compiler_params=pltpu.CompilerParams(needs_layout_passes=False),
    )
    return f(entity, src_ids0, edge_src0)


def _sc2_body(entity, src_ids1, edge_src1, agg, msg_g, aggm_g,
              src_tab, echunk, idxbuf, rows1, rows2, sem1, sem2):
    wid = lax.axis_index("s") * 2 + lax.axis_index("c")
    pltpu.sync_copy(src_ids1, src_tab)

    def chunk_body(k, carry):
        c = wid + k * NW

        @pl.when(c < NCHUNK1)
        def _():
            base = c * CH_E
            pltpu.sync_copy(edge_src1.at[pl.ds(base, CH_E)], echunk)
            for j in range(CH_E // 16):
                ev = echunk[pl.ds(j * 16, 16)]
                idxbuf[pl.ds(j * 16, 16)] = plsc.load_gather(src_tab, [ev])
            cp1 = pltpu.async_copy(entity.at[idxbuf], rows1, sem1)
            cp2 = pltpu.async_copy(agg.at[echunk], rows2, sem2)
            cp1.wait()
            cp2.wait()
            pltpu.sync_copy(rows1, msg_g.at[pl.ds(base, CH_E)])
            pltpu.sync_copy(rows2, aggm_g.at[pl.ds(base, CH_E)])
        return carry

    lax.fori_loop(0, KMAX1, chunk_body, 0)


def _sc2(entity, src_ids1, edge_src1, agg):
    mesh = plsc.VectorSubcoreMesh(core_axis_name="c", subcore_axis_name="s")
    f = pl.kernel(
        _sc2_body,
        out_type=(
            jax.ShapeDtypeStruct((E1, HIDDEN), jnp.float32),
            jax.ShapeDtypeStruct((E1, HIDDEN), jnp.float32),
        ),
        mesh=mesh,
        scratch_types=[
            pltpu.VMEM((N_SRC1,), jnp.int32),
            pltpu.VMEM((CH_E,), jnp.int32),
            pltpu.VMEM((CH_E,), jnp.int32),
            pltpu.VMEM((CH_E, HIDDEN), jnp.float32),
            pltpu.VMEM((CH_E, HIDDEN), jnp.float32),
            pltpu.SemaphoreType.DMA,
            pltpu.SemaphoreType.DMA,
        ],
        compiler_params=pltpu.CompilerParams(needs_layout_passes=False),
    )
    return f(entity, src_ids1, edge_src1, agg)


def _signed_onehot(et, n):
    r = et % NUM_RELS
    sign = jnp.where(et >= NUM_RELS, -1.0, 1.0).astype(jnp.float32)
    oneh = (lax.broadcasted_iota(jnp.int32, (n, NUM_RELS), 1) == r[:, None])
    return oneh.astype(jnp.float32) * sign[:, None]


def _tc1_body(aggE_ref, et_ref, rel_ref, out_ref):
    ne = TILE0 * DEG
    et = et_ref[0, 0, :]
    oneh = _signed_onehot(et, ne)
    cnt = oneh.reshape(TILE0, DEG, NUM_RELS).sum(axis=1)
    aggR = jnp.dot(cnt, rel_ref[...], preferred_element_type=jnp.float32)
    out_ref[...] = (aggE_ref[...] + aggR) * (1.0 / DEG)


def _tc1(aggE, etype0_r, relation, interpret=False):
    grid = N_DST0 // TILE0
    return pl.pallas_call(
        _tc1_body,
        grid=(grid,),
        in_specs=[
            pl.BlockSpec((TILE0, HIDDEN), lambda i: (i, 0)),
            pl.BlockSpec((1, 1, TILE0 * DEG), lambda i: (i, 0, 0)),
            pl.BlockSpec((NUM_RELS, HIDDEN), lambda i: (0, 0)),
        ],
        out_specs=pl.BlockSpec((TILE0, HIDDEN), lambda i: (i, 0)),
        out_shape=jax.ShapeDtypeStruct((N_DST0, HIDDEN), jnp.float32),
        interpret=interpret,
    )(aggE, etype0_r, relation)


def _attn_pool(x, w_ref, b_ref):
    ne = TILE1 * DEG
    w = w_ref[...]
    b = b_ref[0:1, :]
    m1 = jnp.dot(jax.nn.relu(x), w, preferred_element_type=jnp.float32) + b
    xm = x.reshape(TILE1, DEG, HIDDEN).mean(axis=1)
    m2 = jnp.dot(jax.nn.relu(xm), w, preferred_element_type=jnp.float32) + b
    s1 = m1.mean(axis=-1).reshape(TILE1, DEG)
    s2 = m2.mean(axis=-1)[:, None]
    mx = jnp.maximum(s1.max(axis=1), s2[:, 0])
    e1 = jnp.exp(s1 - mx[:, None])
    e2 = jnp.exp(s2 - mx[:, None])
    z = e1.sum(axis=1) + e2[:, 0]
    w1 = e1 / z[:, None]
    w2 = e2 / z[:, None]
    pooled = (w1[:, :, None] * m1.reshape(TILE1, DEG, NUM_TYPES)).sum(axis=1)
    return pooled + w2 * m2


def _tc2_body(msg_ref, aggm_ref, et_ref, rel_ref, w_ref, b_ref, out_ref):
    ne = TILE1 * DEG
    et = et_ref[0, 0, :]
    rel2 = jnp.dot(_signed_onehot(et, ne), rel_ref[...],
                   preferred_element_type=jnp.float32)
    p1 = _attn_pool(msg_ref[...] + rel2, w_ref, b_ref)
    p2 = _attn_pool(aggm_ref[...] + rel2, w_ref, b_ref)
    out_ref[...] = jax.nn.sigmoid(BETA * p1 + (1.0 - BETA) * p2)


def _tc2(msg_g, aggm_g, etype1_r, relation, fc_W, fc_b2, interpret=False):
    grid = N_DST1 // TILE1
    ne = TILE1 * DEG
    return pl.pallas_call(
        _tc2_body,
        grid=(grid,),
        in_specs=[
            pl.BlockSpec((ne, HIDDEN), lambda i: (i, 0)),
            pl.BlockSpec((ne, HIDDEN), lambda i: (i, 0)),
            pl.BlockSpec((1, 1, ne), lambda i: (i, 0, 0)),
            pl.BlockSpec((NUM_RELS, HIDDEN), lambda i: (0, 0)),
            pl.BlockSpec((HIDDEN, NUM_TYPES), lambda i: (0, 0)),
            pl.BlockSpec((8, NUM_TYPES), lambda i: (0, 0)),
        ],
        out_specs=pl.BlockSpec((TILE1, NUM_TYPES), lambda i: (i, 0)),
        out_shape=jax.ShapeDtypeStruct((N_DST1, NUM_TYPES), jnp.float32),
        interpret=interpret,
    )(msg_g, aggm_g, etype1_r, relation, fc_W, fc_b2)


def kernel(entity, relation, fc_W, fc_b,
           src_ids0, edge_src0, etype0, src_ids1, edge_src1, etype1):
    aggE = _sc1(entity, src_ids0, edge_src0)
    agg = _tc1(aggE, etype0.reshape(N_DST0 // TILE0, 1, TILE0 * DEG), relation)
    msg_g, aggm_g = _sc2(entity, src_ids1, edge_src1, agg)
    fc_b2 = jnp.broadcast_to(fc_b[None, :], (8, NUM_TYPES))
    out = _tc2(msg_g, aggm_g,
               etype1.reshape(N_DST1 // TILE1, 1, TILE1 * DEG),
               relation, fc_W, fc_b2)
    return out
